# Initial kernel scaffold; baseline (speedup 1.0000x reference)
import jax
import jax.numpy as jnp
from jax.experimental import pallas as pl


def kernel(inputMatrix, dims, thresholds, lut, selectionMatrix, treeDesMat):
    raise NotImplementedError("write your pallas kernel here")


# speedup vs baseline: 1.5031x; 1.5031x over previous
import functools

import jax
import jax.numpy as jnp
from jax import lax
from jax.experimental import pallas as pl
from jax.experimental.pallas import tpu as pltpu
from jax.experimental.pallas import tpu_sc as plsc

C = 128
K = 16
NUM_CORES = 2
NUM_SUBCORES = 16
NUM_WORKERS = NUM_CORES * NUM_SUBCORES


def _bf16_round(v):
    u = plsc.bitcast(v, jnp.int32)
    r = u + 0x7FFF + (lax.shift_right_logical(u, 16) & 1)
    return plsc.bitcast(r & jnp.int32(-65536), jnp.float32)


def _sc_encode_body(n_tokens, in_features, x_hbm, dims_hbm, thr_hbm, idx_hbm,
                    dims_v, thr_v, row_v, idx_v, in_sem):
    tpw = n_tokens // NUM_WORKERS
    wid = lax.axis_index("s") * NUM_CORES + lax.axis_index("c")
    tok0 = wid * tpw

    pltpu.sync_copy(dims_hbm, dims_v)
    pltpu.sync_copy(thr_hbm, thr_v)

    def row_copy(t):
        slot = lax.rem(t, 2) * in_features
        return pltpu.make_async_copy(
            x_hbm.at[tok0 + t], row_v.at[pl.ds(slot, in_features)], in_sem)

    row_copy(0).start()

    iota = lax.iota(jnp.int32, 16)
    ones16 = jnp.ones((16,), jnp.int32)

    def token_body(t, carry):
        @pl.when(t + 1 < tpw)
        def _():
            row_copy(t + 1).start()
        row_copy(t).wait()
        base = lax.rem(t, 2) * in_features
        for g in range(C // 16):
            cvec = iota + (g * 16)
            i0 = dims_v[pl.ds(g * 16, 16)]
            i1 = dims_v[pl.ds(C + g * 16, 16)]
            i2 = dims_v[pl.ds(2 * C + g * 16, 16)]
            i3 = dims_v[pl.ds(3 * C + g * 16, 16)]
            v0 = _bf16_round(plsc.load_gather(row_v, [i0 + base]))
            t0 = thr_v[pl.ds(g * 16, 16)]
            b0 = jnp.where(v0 > t0, ones16, 0)
            r1 = 1 + b0
            t1 = plsc.load_gather(thr_v, [r1 * C + cvec])
            v1 = _bf16_round(plsc.load_gather(row_v, [i1 + base]))
            b1 = jnp.where(v1 > t1, ones16, 0)
            r2 = 3 + 2 * b0 + b1
            t2 = plsc.load_gather(thr_v, [r2 * C + cvec])
            v2 = _bf16_round(plsc.load_gather(row_v, [i2 + base]))
            b2 = jnp.where(v2 > t2, ones16, 0)
            r3 = 7 + 4 * b0 + 2 * b1 + b2
            t3 = plsc.load_gather(thr_v, [r3 * C + cvec])
            v3 = _bf16_round(plsc.load_gather(row_v, [i3 + base]))
            b3 = jnp.where(v3 > t3, ones16, 0)
            leaf = 8 * b0 + 4 * b1 + 2 * b2 + b3
            idx_v[pl.ds(t * C + g * 16, 16)] = leaf.astype(jnp.float32)
        return carry

    lax.fori_loop(0, tpw, token_body, 0)
    pltpu.sync_copy(idx_v, idx_hbm.at[pl.ds(tok0 * C, tpw * C)])


def _sc_encode(x, dims_r, thr_r):
    n_tokens, in_features = x.shape
    tpw = n_tokens // NUM_WORKERS
    mesh = plsc.VectorSubcoreMesh(core_axis_name="c", subcore_axis_name="s",
                                  num_cores=NUM_CORES,
                                  num_subcores=NUM_SUBCORES)
    body = functools.partial(_sc_encode_body, n_tokens, in_features)
    fn = pl.kernel(
        body,
        out_type=jax.ShapeDtypeStruct((n_tokens * C,), jnp.float32),
        mesh=mesh,
        scratch_types=[
            pltpu.VMEM((4 * C,), jnp.int32),
            pltpu.VMEM((15 * C,), jnp.float32),
            pltpu.VMEM((2 * in_features,), jnp.float32),
            pltpu.VMEM((tpw * C,), jnp.float32),
            pltpu.SemaphoreType.DMA,
        ],
        compiler_params=pltpu.CompilerParams(needs_layout_passes=False),
    )
    return fn(x, dims_r, thr_r)


def _tc_decode_body(idx_ref, r_ref, l_ref, o_ref):
    bm = idx_ref.shape[0]
    idxb = idx_ref[...].astype(jnp.bfloat16)
    rep = lax.dot_general(idxb, r_ref[...], (((1,), (0,)), ((), ())),
                          preferred_element_type=jnp.float32)
    kv = lax.broadcasted_iota(jnp.int32, (bm, C * K), 1) & (K - 1)
    e = (rep == kv.astype(jnp.float32)).astype(jnp.bfloat16)
    o_ref[...] = lax.dot_general(e, l_ref[...], (((1,), (1,)), ((), ())),
                                 preferred_element_type=jnp.float32)


def _tc_decode(idxm, rep_mat, lut_flat):
    n_tokens = idxm.shape[0]
    out_features = lut_flat.shape[0]
    bm = 256
    grid = (n_tokens // bm,)
    return pl.pallas_call(
        _tc_decode_body,
        grid=grid,
        in_specs=[
            pl.BlockSpec((bm, C), lambda i: (i, 0)),
            pl.BlockSpec((C, C * K), lambda i: (0, 0)),
            pl.BlockSpec((out_features, C * K), lambda i: (0, 0)),
        ],
        out_specs=pl.BlockSpec((bm, out_features), lambda i: (i, 0)),
        out_shape=jax.ShapeDtypeStruct((n_tokens, out_features), jnp.float32),
    )(idxm, rep_mat, lut_flat)


def kernel(inputMatrix, dims, thresholds, lut, selectionMatrix, treeDesMat):
    n_tokens, in_features = inputMatrix.shape
    out_features = lut.shape[0]

    dims_r = dims.reshape(C, 4).T.reshape(-1)
    thr_r = thresholds.reshape(C, 15).T.reshape(-1)

    idx_flat = _sc_encode(inputMatrix, dims_r, thr_r)
    idxm = idx_flat.reshape(n_tokens, C)

    rep_mat = (lax.broadcasted_iota(jnp.int32, (C, C * K), 1) // K ==
               lax.broadcasted_iota(jnp.int32, (C, C * K), 0)
               ).astype(jnp.bfloat16)
    lut_flat = lut.reshape(out_features, C * K).astype(jnp.bfloat16)
    return _tc_decode(idxm, rep_mat, lut_flat)
